# baseline (speedup 1.0000x reference)
import jax
import jax.numpy as jnp
from jax import lax
from jax.experimental import pallas as pl
from jax.experimental.pallas import tpu as pltpu


def _round_up(x, m):
    return (x + m - 1) // m * m


def _fused_kernel(x_ref, c_ref, o_ref, *, hid, n_classes):
    in_f = x_ref.shape[1]
    w2_rows = (n_classes * hid) // in_f

    h = lax.dot_general(
        c_ref[0:hid, :], x_ref[...],
        dimension_numbers=(((1,), (1,)), ((), ())),
        preferred_element_type=jnp.float32,
    )
    b1c = c_ref[c_ref.shape[0] - 1:c_ref.shape[0], 0:hid].reshape(hid, 1)
    h = jnp.maximum(h + b1c, 0.0)

    b2c = (c_ref[c_ref.shape[0] - 1:c_ref.shape[0], hid:hid + n_classes]
           .reshape(n_classes, 1))
    w2v = c_ref[hid:hid + w2_rows, :].reshape(n_classes, hid)
    logits = jnp.dot(w2v, h, preferred_element_type=jnp.float32) + b2c

    m = jnp.max(logits, axis=0, keepdims=True)
    e = jnp.exp(logits - m)
    denom = jnp.sum(e, axis=0, keepdims=True)
    o_ref[...] = e * (1.0 / denom)


def kernel(x, w1, b1, w2, b2):
    import functools

    B, in_f = x.shape
    hid = w1.shape[0]
    C = w2.shape[0]

    assert (C * hid) % in_f == 0, "w2 does not flat-pack into container rows"
    w2_rows = (C * hid) // in_f
    w2p = w2.reshape(w2_rows, in_f)
    bias_row = jnp.pad(jnp.concatenate([b1, b2])[None, :],
                       ((0, 0), (0, in_f - hid - C)))
    cont = jnp.concatenate([w1, w2p, bias_row], axis=0)
    rows = hid + w2_rows + 1

    tb = 1024 if B % 1024 == 0 else B
    grid = (B // tb,)

    x_tile = _round_up(tb, 8) * _round_up(in_f, 128) * 4
    o_tile = _round_up(C, 8) * _round_up(tb, 128) * 4
    c_tile = _round_up(rows, 8) * _round_up(in_f, 128) * 4
    vmem_limit_bytes = int(min(
        max(2 * (x_tile + o_tile) + c_tile + (6 << 20), 32 << 20), 100 << 20))

    out_cb = pl.pallas_call(
        functools.partial(_fused_kernel, hid=hid, n_classes=C),
        out_shape=jax.ShapeDtypeStruct((C, B), jnp.float32),
        grid=grid,
        in_specs=[
            pl.BlockSpec((tb, in_f), lambda i: (i, 0)),
            pl.BlockSpec((rows, in_f), lambda i: (0, 0)),
        ],
        out_specs=pl.BlockSpec((C, tb), lambda i: (0, i)),
        compiler_params=pltpu.CompilerParams(
            dimension_semantics=("parallel",),
            vmem_limit_bytes=vmem_limit_bytes,
        ),
        cost_estimate=pl.CostEstimate(
            flops=2 * B * (in_f * hid + hid * C),
            transcendentals=B * C,
            bytes_accessed=4 * (B * in_f + B * C + hid * in_f + C * hid),
        ),
    )(x, cont)

    return out_cb.T


# speedup vs baseline: 1.0121x; 1.0121x over previous
import jax
import jax.numpy as jnp
from jax import lax
from jax.experimental import pallas as pl
from jax.experimental.pallas import tpu as pltpu


def _round_up(x, m):
    return (x + m - 1) // m * m


def _fused_kernel(x_ref, c_ref, o_ref, *, hid, n_classes):
    w2_lo = _round_up(hid, 8)

    h = lax.dot_general(
        c_ref[0:hid, :], x_ref[...],
        dimension_numbers=(((1,), (1,)), ((), ())),
        preferred_element_type=jnp.float32,
    )
    b1c = c_ref[c_ref.shape[0] - 1:c_ref.shape[0], 0:hid].reshape(hid, 1)
    h = jnp.maximum(h + b1c, 0.0)

    b2c = (c_ref[c_ref.shape[0] - 1:c_ref.shape[0], hid:hid + n_classes]
           .reshape(n_classes, 1))
    logits = jnp.dot(c_ref[w2_lo:w2_lo + n_classes, 0:hid], h,
                     preferred_element_type=jnp.float32) + b2c

    m = jnp.max(logits, axis=0, keepdims=True)
    e = jnp.exp(logits - m)
    denom = jnp.sum(e, axis=0, keepdims=True)
    o_ref[...] = e * (1.0 / denom)


def kernel(x, w1, b1, w2, b2):
    import functools

    B, in_f = x.shape
    hid = w1.shape[0]
    C = w2.shape[0]

    w2_rows = _round_up(C, 8)
    w2p = jnp.pad(w2, ((0, w2_rows - C), (0, in_f - hid)))
    bias_row = jnp.pad(jnp.concatenate([b1, b2])[None, :],
                       ((0, 0), (0, in_f - hid - C)))
    cont = jnp.concatenate([w1, w2p, bias_row], axis=0)
    rows = hid + w2_rows + 1

    tb = 1024 if B % 1024 == 0 else B
    grid = (B // tb,)

    x_tile = _round_up(tb, 8) * _round_up(in_f, 128) * 4
    o_tile = _round_up(C, 8) * _round_up(tb, 128) * 4
    c_tile = _round_up(rows, 8) * _round_up(in_f, 128) * 4
    vmem_limit_bytes = int(min(
        max(2 * (x_tile + o_tile) + c_tile + (6 << 20), 32 << 20), 100 << 20))

    out_cb = pl.pallas_call(
        functools.partial(_fused_kernel, hid=hid, n_classes=C),
        out_shape=jax.ShapeDtypeStruct((C, B), jnp.float32),
        grid=grid,
        in_specs=[
            pl.BlockSpec((tb, in_f), lambda i: (i, 0)),
            pl.BlockSpec((rows, in_f), lambda i: (0, 0)),
        ],
        out_specs=pl.BlockSpec((C, tb), lambda i: (0, i)),
        compiler_params=pltpu.CompilerParams(
            dimension_semantics=("parallel",),
            vmem_limit_bytes=vmem_limit_bytes,
        ),
        cost_estimate=pl.CostEstimate(
            flops=2 * B * (in_f * hid + hid * C),
            transcendentals=B * C,
            bytes_accessed=4 * (B * in_f + B * C + hid * in_f + C * hid),
        ),
    )(x, cont)

    return out_cb.T
